# baseline (speedup 1.0000x reference)
import jax
import jax.numpy as jnp
from jax import lax
from jax.experimental import pallas as pl
from jax.experimental.pallas import tpu as pltpu
from jax.experimental.pallas import tpu_sc as plsc

N_NODES = 10000
N_INC = 320000
D = 128

NC = 2
NS = 16
L = 16
NW = NC * NS
GCH = 64
NB = 2
NG = 80
STEPS = NB * NG
P_INC = NW * STEPS * GCH
CCH = 128
CSTEPS = P_INC // (NW * CCH)
ROWS_PAD = 10112
RPT = ROWS_PAD // NS
BM = 1264


def _linear_body(x_ref, wt_ref, b_ref, o_ref):
    o_ref[...] = (
        jnp.dot(x_ref[...], wt_ref[...], preferred_element_type=jnp.float32)
        + b_ref[...]
    )


def _linear(x, wt, b):
    m = x.shape[0]
    return pl.pallas_call(
        _linear_body,
        grid=(m // BM,),
        in_specs=[
            pl.BlockSpec((BM, D), lambda i: (i, 0)),
            pl.BlockSpec((D, D), lambda i: (0, 0)),
            pl.BlockSpec((1, D), lambda i: (0, 0)),
        ],
        out_specs=pl.BlockSpec((BM, D), lambda i: (i, 0)),
        out_shape=jax.ShapeDtypeStruct((m, D), jnp.float32),
    )(x, wt, b)


def _mid_body(p_ref, c_ref, wt_ref, b_ref, o_ref):
    s = p_ref[0] + p_ref[1]
    cnt = c_ref[0, :, 0:1] + c_ref[1, :, 0:1]
    mean = jnp.maximum(s / jnp.maximum(cnt, 1.0), 0.0)
    o_ref[...] = (
        jnp.dot(mean, wt_ref[...], preferred_element_type=jnp.float32)
        + b_ref[...]
    )


def _mid(p, c, wt, b):
    return pl.pallas_call(
        _mid_body,
        grid=(ROWS_PAD // BM,),
        in_specs=[
            pl.BlockSpec((NC, BM, D), lambda i: (0, i, 0)),
            pl.BlockSpec((NC, BM, L), lambda i: (0, i, 0)),
            pl.BlockSpec((D, D), lambda i: (0, 0)),
            pl.BlockSpec((1, D), lambda i: (0, 0)),
        ],
        out_specs=pl.BlockSpec((BM, D), lambda i: (i, 0)),
        out_shape=jax.ShapeDtypeStruct((ROWS_PAD, D), jnp.float32),
    )(p, c, wt, b)


def _final_body(q_ref, c_ref, o_ref):
    s = q_ref[0] + q_ref[1]
    cnt = c_ref[0, :, 0:1] + c_ref[1, :, 0:1]
    o_ref[...] = jnp.maximum(s / jnp.maximum(cnt, 1.0), 0.0)


def _final(q, c):
    return pl.pallas_call(
        _final_body,
        grid=(ROWS_PAD // BM,),
        in_specs=[
            pl.BlockSpec((NC, BM, D), lambda i: (0, i, 0)),
            pl.BlockSpec((NC, BM, L), lambda i: (0, i, 0)),
        ],
        out_specs=pl.BlockSpec((BM, D), lambda i: (i, 0)),
        out_shape=jax.ShapeDtypeStruct((ROWS_PAD, D), jnp.float32),
    )(q, c)


_MESH = plsc.VectorSubcoreMesh(core_axis_name="c", subcore_axis_name="s")
_SC_PARAMS = pltpu.CompilerParams(use_tc_tiling_on_sc=False)


def _counts_body(gidx_hbm, sidx_hbm, ecnt_hbm, vcnt_hbm,
                 gidx_v, sidx_v, ones_v, zcnt_v, ecnt_sh, vcnt_sh):
    co = lax.axis_index("c")
    s = lax.axis_index("s")
    w = co * NS + s
    pltpu.sync_copy(gidx_hbm.at[w], gidx_v)
    pltpu.sync_copy(sidx_hbm.at[w], sidx_v)

    @pl.loop(0, CCH)
    def _(i):
        ones_v[i, :] = jnp.ones((L,), jnp.float32)

    @pl.loop(0, RPT)
    def _(i):
        zcnt_v[i, :] = jnp.zeros((L,), jnp.float32)

    base = s * RPT
    pltpu.sync_copy(zcnt_v, ecnt_sh.at[pl.ds(base, RPT)])
    pltpu.sync_copy(zcnt_v, vcnt_sh.at[pl.ds(base, RPT)])

    plsc.subcore_barrier()

    @pl.loop(0, CSTEPS)
    def _(j):
        pltpu.sync_copy(ones_v, ecnt_sh.at[sidx_v.at[j]], add=True)
        pltpu.sync_copy(ones_v, vcnt_sh.at[gidx_v.at[j]], add=True)

    plsc.subcore_barrier()

    pltpu.sync_copy(ecnt_sh.at[pl.ds(base, RPT)], ecnt_hbm.at[co, pl.ds(base, RPT)])
    pltpu.sync_copy(vcnt_sh.at[pl.ds(base, RPT)], vcnt_hbm.at[co, pl.ds(base, RPT)])


def _counts(gidx, sidx):
    f = pl.kernel(
        _counts_body,
        out_type=[
            jax.ShapeDtypeStruct((NC, ROWS_PAD, L), jnp.float32),
            jax.ShapeDtypeStruct((NC, ROWS_PAD, L), jnp.float32),
        ],
        mesh=_MESH,
        scratch_types=[
            pltpu.VMEM((CSTEPS, CCH), jnp.int32),
            pltpu.VMEM((CSTEPS, CCH), jnp.int32),
            pltpu.VMEM((CCH, L), jnp.float32),
            pltpu.VMEM((RPT, L), jnp.float32),
            pltpu.VMEM_SHARED((ROWS_PAD, L), jnp.float32),
            pltpu.VMEM_SHARED((ROWS_PAD, L), jnp.float32),
        ],
        compiler_params=_SC_PARAMS,
    )
    return f(gidx, sidx)


def _agg_body(h_hbm, gidx_hbm, sidx_hbm, out_hbm,
              gidx_v, sidx_v, rows_v, zbuf_v, acc_sh, gsem, ssem):
    co = lax.axis_index("c")
    s = lax.axis_index("s")
    w = co * NS + s
    pltpu.sync_copy(gidx_hbm.at[w], gidx_v)
    pltpu.sync_copy(sidx_hbm.at[w], sidx_v)

    @pl.loop(0, 40)
    def _(i):
        for jj in range(D // L):
            zbuf_v[i, pl.ds(jj * L, L)] = jnp.zeros((L,), jnp.float32)

    base = s * RPT

    @pl.loop(0, 15)
    def _(k):
        pltpu.sync_copy(zbuf_v, acc_sh.at[pl.ds(base + k * 40, 40)])

    pltpu.sync_copy(zbuf_v.at[pl.ds(0, RPT - 600)],
                    acc_sh.at[pl.ds(base + 600, RPT - 600)])

    plsc.subcore_barrier()

    for b in range(NB):
        pltpu.async_copy(h_hbm.at[gidx_v.at[b]], rows_v.at[b], gsem.at[b])

    @pl.loop(0, NG)
    def _(g):
        j0 = g * NB
        for b in range(NB):
            pltpu.make_async_copy(
                h_hbm.at[gidx_v.at[j0 + b]], rows_v.at[b], gsem.at[b]
            ).wait()
            pltpu.async_copy(
                rows_v.at[b], acc_sh.at[sidx_v.at[j0 + b]], ssem.at[b],
                add=True)
        for b in range(NB):
            pltpu.make_async_copy(
                rows_v.at[b], acc_sh.at[sidx_v.at[j0 + b]], ssem.at[b]
            ).wait()

            @pl.when(g < NG - 1)
            def _():
                pltpu.async_copy(
                    h_hbm.at[gidx_v.at[j0 + NB + b]], rows_v.at[b], gsem.at[b])

    plsc.subcore_barrier()

    pltpu.sync_copy(acc_sh.at[pl.ds(base, RPT)],
                    out_hbm.at[co, pl.ds(base, RPT)])


def _agg(h, gidx, sidx):
    f = pl.kernel(
        _agg_body,
        out_type=jax.ShapeDtypeStruct((NC, ROWS_PAD, D), jnp.float32),
        mesh=_MESH,
        scratch_types=[
            pltpu.VMEM((STEPS, GCH), jnp.int32),
            pltpu.VMEM((STEPS, GCH), jnp.int32),
            pltpu.VMEM((NB, GCH, D), jnp.float32),
            pltpu.VMEM((40, D), jnp.float32),
            pltpu.VMEM_SHARED((ROWS_PAD, D), jnp.float32),
            pltpu.SemaphoreType.DMA((NB,)),
            pltpu.SemaphoreType.DMA((NB,)),
        ],
        compiler_params=_SC_PARAMS,
    )
    return f(h, gidx, sidx)


def kernel(x, hyperedge_index, W_v2e, b_v2e, W_e2v, b_e2v):
    nidx = hyperedge_index[0].astype(jnp.int32)
    eidx = hyperedge_index[1].astype(jnp.int32)
    pad = P_INC - N_INC
    fill = jnp.full((pad,), N_NODES, jnp.int32)
    nidx_p = jnp.concatenate([nidx, fill])
    eidx_p = jnp.concatenate([eidx, fill])
    nidx_g = nidx_p.reshape(NW, STEPS, GCH)
    eidx_g = eidx_p.reshape(NW, STEPS, GCH)
    nidx_c = nidx_p.reshape(NW, CSTEPS, CCH)
    eidx_c = eidx_p.reshape(NW, CSTEPS, CCH)
    x_p = jnp.pad(x, ((0, ROWS_PAD - N_NODES), (0, 0)))

    h = _linear(x_p, W_v2e.T, b_v2e.reshape(1, D))
    ecnt, vcnt = _counts(nidx_c, eidx_c)
    esum = _agg(h, nidx_g, eidx_g)
    e = _mid(esum, ecnt, W_e2v.T, b_e2v.reshape(1, D))
    vsum = _agg(e, eidx_g, nidx_g)
    out = _final(vsum, vcnt)
    return out[:N_NODES]


# speedup vs baseline: 2.3409x; 2.3409x over previous
import jax
import jax.numpy as jnp
from jax import lax
from jax.experimental import pallas as pl
from jax.experimental.pallas import tpu as pltpu
from jax.experimental.pallas import tpu_sc as plsc

N_NODES = 10000
N_INC = 320000
D = 128

NC = 2
NS = 16
L = 16
NW = NC * NS
GCH = 64
NB = 2
NG = 80
STEPS = NB * NG
P_INC = NW * STEPS * GCH
CCH = 128
CSTEPS = P_INC // (NW * CCH)
ROWS_PAD = 10112
RPT = ROWS_PAD // NS
BM = 1264


def _linear_body(x_ref, wt_ref, b_ref, o_ref):
    o_ref[...] = (
        jnp.dot(x_ref[...], wt_ref[...], preferred_element_type=jnp.float32)
        + b_ref[...]
    )


def _linear(x, wt, b):
    m = x.shape[0]
    return pl.pallas_call(
        _linear_body,
        grid=(m // BM,),
        in_specs=[
            pl.BlockSpec((BM, D), lambda i: (i, 0)),
            pl.BlockSpec((D, D), lambda i: (0, 0)),
            pl.BlockSpec((1, D), lambda i: (0, 0)),
        ],
        out_specs=pl.BlockSpec((BM, D), lambda i: (i, 0)),
        out_shape=jax.ShapeDtypeStruct((m, D), jnp.float32),
    )(x, wt, b)


def _mid_body(p_ref, c_ref, wt_ref, b_ref, o_ref):
    s = p_ref[0] + p_ref[1]
    cnt = c_ref[0, :, 0:1] + c_ref[1, :, 0:1]
    mean = jnp.maximum(s / jnp.maximum(cnt, 1.0), 0.0)
    o_ref[...] = (
        jnp.dot(mean, wt_ref[...], preferred_element_type=jnp.float32)
        + b_ref[...]
    )


def _mid(p, c, wt, b):
    return pl.pallas_call(
        _mid_body,
        grid=(ROWS_PAD // BM,),
        in_specs=[
            pl.BlockSpec((NC, BM, D), lambda i: (0, i, 0)),
            pl.BlockSpec((NC, BM, L), lambda i: (0, i, 0)),
            pl.BlockSpec((D, D), lambda i: (0, 0)),
            pl.BlockSpec((1, D), lambda i: (0, 0)),
        ],
        out_specs=pl.BlockSpec((BM, D), lambda i: (i, 0)),
        out_shape=jax.ShapeDtypeStruct((ROWS_PAD, D), jnp.float32),
    )(p, c, wt, b)


def _final_body(q_ref, c_ref, o_ref):
    s = q_ref[0] + q_ref[1]
    cnt = c_ref[0, :, 0:1] + c_ref[1, :, 0:1]
    o_ref[...] = jnp.maximum(s / jnp.maximum(cnt, 1.0), 0.0)


def _final(q, c):
    return pl.pallas_call(
        _final_body,
        grid=(ROWS_PAD // BM,),
        in_specs=[
            pl.BlockSpec((NC, BM, D), lambda i: (0, i, 0)),
            pl.BlockSpec((NC, BM, L), lambda i: (0, i, 0)),
        ],
        out_specs=pl.BlockSpec((BM, D), lambda i: (i, 0)),
        out_shape=jax.ShapeDtypeStruct((ROWS_PAD, D), jnp.float32),
    )(q, c)


_MESH = plsc.VectorSubcoreMesh(core_axis_name="c", subcore_axis_name="s")
_SC_PARAMS = pltpu.CompilerParams(use_tc_tiling_on_sc=False)


def _counts_body(gidx_hbm, sidx_hbm, ecnt_hbm, vcnt_hbm,
                 gidx_v, sidx_v, ones_v, zcnt_v, ecnt_sh, vcnt_sh):
    co = lax.axis_index("c")
    s = lax.axis_index("s")
    w = co * NS + s
    pltpu.sync_copy(gidx_hbm.at[w], gidx_v)
    pltpu.sync_copy(sidx_hbm.at[w], sidx_v)

    @pl.loop(0, CCH)
    def _(i):
        ones_v[i, :] = jnp.ones((L,), jnp.float32)

    @pl.loop(0, RPT)
    def _(i):
        zcnt_v[i, :] = jnp.zeros((L,), jnp.float32)

    base = s * RPT
    pltpu.sync_copy(zcnt_v, ecnt_sh.at[pl.ds(base, RPT)])
    pltpu.sync_copy(zcnt_v, vcnt_sh.at[pl.ds(base, RPT)])

    plsc.subcore_barrier()

    @pl.loop(0, CSTEPS)
    def _(j):
        pltpu.sync_copy(ones_v, ecnt_sh.at[sidx_v.at[j]], add=True)
        pltpu.sync_copy(ones_v, vcnt_sh.at[gidx_v.at[j]], add=True)

    plsc.subcore_barrier()

    pltpu.sync_copy(ecnt_sh.at[pl.ds(base, RPT)], ecnt_hbm.at[co, pl.ds(base, RPT)])
    pltpu.sync_copy(vcnt_sh.at[pl.ds(base, RPT)], vcnt_hbm.at[co, pl.ds(base, RPT)])


def _counts(gidx, sidx):
    f = pl.kernel(
        _counts_body,
        out_type=[
            jax.ShapeDtypeStruct((NC, ROWS_PAD, L), jnp.float32),
            jax.ShapeDtypeStruct((NC, ROWS_PAD, L), jnp.float32),
        ],
        mesh=_MESH,
        scratch_types=[
            pltpu.VMEM((CSTEPS, CCH), jnp.int32),
            pltpu.VMEM((CSTEPS, CCH), jnp.int32),
            pltpu.VMEM((CCH, L), jnp.float32),
            pltpu.VMEM((RPT, L), jnp.float32),
            pltpu.VMEM_SHARED((ROWS_PAD, L), jnp.float32),
            pltpu.VMEM_SHARED((ROWS_PAD, L), jnp.float32),
        ],
        compiler_params=_SC_PARAMS,
    )
    return f(gidx, sidx)


def _agg_body(h_hbm, gidx_hbm, sidx_hbm, out_hbm,
              gidx_v, sidx_v, rows_v, zbuf_v, acc_sh, gsem, ssem):
    co = lax.axis_index("c")
    s = lax.axis_index("s")
    w = co * NS + s
    pltpu.sync_copy(gidx_hbm.at[w], gidx_v)
    pltpu.sync_copy(sidx_hbm.at[w], sidx_v)

    @pl.loop(0, 40)
    def _(i):
        for jj in range(D // L):
            zbuf_v[i, pl.ds(jj * L, L)] = jnp.zeros((L,), jnp.float32)

    base = s * RPT

    @pl.loop(0, 15)
    def _(k):
        pltpu.sync_copy(zbuf_v, acc_sh.at[pl.ds(base + k * 40, 40)])

    pltpu.sync_copy(zbuf_v.at[pl.ds(0, RPT - 600)],
                    acc_sh.at[pl.ds(base + 600, RPT - 600)])

    plsc.subcore_barrier()

    for b in range(NB):
        pltpu.async_copy(h_hbm.at[gidx_v.at[b]], rows_v.at[b], gsem.at[b])

    @pl.loop(0, NG)
    def _(g):
        j0 = g * NB
        for b in range(NB):
            pltpu.make_async_copy(
                h_hbm.at[gidx_v.at[j0 + b]], rows_v.at[b], gsem.at[b]
            ).wait()
            pltpu.async_copy(
                rows_v.at[b], acc_sh.at[sidx_v.at[j0 + b]], ssem.at[b],
                add=True)
        for b in range(NB):
            pltpu.make_async_copy(
                rows_v.at[b], acc_sh.at[sidx_v.at[j0 + b]], ssem.at[b]
            ).wait()

            @pl.when(g < NG - 1)
            def _():
                pltpu.async_copy(
                    h_hbm.at[gidx_v.at[j0 + NB + b]], rows_v.at[b], gsem.at[b])

    plsc.subcore_barrier()

    pltpu.sync_copy(acc_sh.at[pl.ds(base, RPT)],
                    out_hbm.at[co, pl.ds(base, RPT)])


def _agg(h, gidx, sidx):
    f = pl.kernel(
        _agg_body,
        out_type=jax.ShapeDtypeStruct((NC, ROWS_PAD, D), jnp.float32),
        mesh=_MESH,
        scratch_types=[
            pltpu.VMEM((STEPS, GCH), jnp.int32),
            pltpu.VMEM((STEPS, GCH), jnp.int32),
            pltpu.VMEM((NB, GCH, D), jnp.float32),
            pltpu.VMEM((40, D), jnp.float32),
            pltpu.VMEM_SHARED((ROWS_PAD, D), jnp.float32),
            pltpu.SemaphoreType.DMA((NB,)),
            pltpu.SemaphoreType.DMA((NB,)),
        ],
        compiler_params=_SC_PARAMS,
    )
    return f(h, gidx, sidx)


def kernel(x, hyperedge_index, W_v2e, b_v2e, W_e2v, b_e2v):
    nidx = hyperedge_index[0].astype(jnp.int32)
    eidx = hyperedge_index[1].astype(jnp.int32)
    pad = P_INC - N_INC
    fill = N_NODES + jnp.arange(pad, dtype=jnp.int32) % (ROWS_PAD - N_NODES)
    nidx_p = jnp.concatenate([nidx, fill])
    eidx_p = jnp.concatenate([eidx, fill])
    nidx_g = nidx_p.reshape(NW, STEPS, GCH)
    eidx_g = eidx_p.reshape(NW, STEPS, GCH)
    nidx_c = nidx_p.reshape(NW, CSTEPS, CCH)
    eidx_c = eidx_p.reshape(NW, CSTEPS, CCH)
    x_p = jnp.pad(x, ((0, ROWS_PAD - N_NODES), (0, 0)))

    h = _linear(x_p, W_v2e.T, b_v2e.reshape(1, D))
    ecnt, vcnt = _counts(nidx_c, eidx_c)
    esum = _agg(h, nidx_g, eidx_g)
    e = _mid(esum, ecnt, W_e2v.T, b_e2v.reshape(1, D))
    vsum = _agg(e, eidx_g, nidx_g)
    out = _final(vsum, vcnt)
    return out[:N_NODES]


# speedup vs baseline: 2.8316x; 1.2096x over previous
import jax
import jax.numpy as jnp
from jax import lax
from jax.experimental import pallas as pl
from jax.experimental.pallas import tpu as pltpu
from jax.experimental.pallas import tpu_sc as plsc

N_NODES = 10000
N_INC = 320000
D = 128

NC = 2
NS = 16
L = 16
NW = NC * NS
GCH = 64
NB = 3
NG = 54
STEPS = NB * NG
P_INC = NW * STEPS * GCH
CCH = 128
CSTEPS = P_INC // (NW * CCH)
ROWS_PAD = 10112
RPT = ROWS_PAD // NS
BM = 1264


def _linear_body(x_ref, wt_ref, b_ref, o_ref):
    o_ref[...] = (
        jnp.dot(x_ref[...], wt_ref[...], preferred_element_type=jnp.float32)
        + b_ref[...]
    )


def _linear(x, wt, b):
    m = x.shape[0]
    return pl.pallas_call(
        _linear_body,
        grid=(m // BM,),
        in_specs=[
            pl.BlockSpec((BM, D), lambda i: (i, 0)),
            pl.BlockSpec((D, D), lambda i: (0, 0)),
            pl.BlockSpec((1, D), lambda i: (0, 0)),
        ],
        out_specs=pl.BlockSpec((BM, D), lambda i: (i, 0)),
        out_shape=jax.ShapeDtypeStruct((m, D), jnp.float32),
    )(x, wt, b)


def _mid_body(p_ref, c_ref, wt_ref, b_ref, o_ref):
    s = p_ref[0] + p_ref[1]
    cnt = c_ref[0, :, 0:1] + c_ref[1, :, 0:1]
    mean = jnp.maximum(s / jnp.maximum(cnt, 1.0), 0.0)
    o_ref[...] = (
        jnp.dot(mean, wt_ref[...], preferred_element_type=jnp.float32)
        + b_ref[...]
    )


def _mid(p, c, wt, b):
    return pl.pallas_call(
        _mid_body,
        grid=(ROWS_PAD // BM,),
        in_specs=[
            pl.BlockSpec((NC, BM, D), lambda i: (0, i, 0)),
            pl.BlockSpec((NC, BM, L), lambda i: (0, i, 0)),
            pl.BlockSpec((D, D), lambda i: (0, 0)),
            pl.BlockSpec((1, D), lambda i: (0, 0)),
        ],
        out_specs=pl.BlockSpec((BM, D), lambda i: (i, 0)),
        out_shape=jax.ShapeDtypeStruct((ROWS_PAD, D), jnp.float32),
    )(p, c, wt, b)


def _final_body(q_ref, c_ref, o_ref):
    s = q_ref[0] + q_ref[1]
    cnt = c_ref[0, :, 0:1] + c_ref[1, :, 0:1]
    o_ref[...] = jnp.maximum(s / jnp.maximum(cnt, 1.0), 0.0)


def _final(q, c):
    return pl.pallas_call(
        _final_body,
        grid=(ROWS_PAD // BM,),
        in_specs=[
            pl.BlockSpec((NC, BM, D), lambda i: (0, i, 0)),
            pl.BlockSpec((NC, BM, L), lambda i: (0, i, 0)),
        ],
        out_specs=pl.BlockSpec((BM, D), lambda i: (i, 0)),
        out_shape=jax.ShapeDtypeStruct((ROWS_PAD, D), jnp.float32),
    )(q, c)


_MESH = plsc.VectorSubcoreMesh(core_axis_name="c", subcore_axis_name="s")
_SC_PARAMS = pltpu.CompilerParams(use_tc_tiling_on_sc=False)


def _counts_body(gidx_hbm, sidx_hbm, ecnt_hbm, vcnt_hbm,
                 gidx_v, sidx_v, ones_v, zcnt_v, ecnt_sh, vcnt_sh):
    co = lax.axis_index("c")
    s = lax.axis_index("s")
    w = co * NS + s
    pltpu.sync_copy(gidx_hbm.at[w], gidx_v)
    pltpu.sync_copy(sidx_hbm.at[w], sidx_v)

    @pl.loop(0, CCH)
    def _(i):
        ones_v[i, :] = jnp.ones((L,), jnp.float32)

    @pl.loop(0, RPT)
    def _(i):
        zcnt_v[i, :] = jnp.zeros((L,), jnp.float32)

    base = s * RPT
    pltpu.sync_copy(zcnt_v, ecnt_sh.at[pl.ds(base, RPT)])
    pltpu.sync_copy(zcnt_v, vcnt_sh.at[pl.ds(base, RPT)])

    plsc.subcore_barrier()

    @pl.loop(0, CSTEPS)
    def _(j):
        pltpu.sync_copy(ones_v, ecnt_sh.at[sidx_v.at[j]], add=True)
        pltpu.sync_copy(ones_v, vcnt_sh.at[gidx_v.at[j]], add=True)

    plsc.subcore_barrier()

    pltpu.sync_copy(ecnt_sh.at[pl.ds(base, RPT)], ecnt_hbm.at[co, pl.ds(base, RPT)])
    pltpu.sync_copy(vcnt_sh.at[pl.ds(base, RPT)], vcnt_hbm.at[co, pl.ds(base, RPT)])


def _counts(gidx, sidx):
    f = pl.kernel(
        _counts_body,
        out_type=[
            jax.ShapeDtypeStruct((NC, ROWS_PAD, L), jnp.float32),
            jax.ShapeDtypeStruct((NC, ROWS_PAD, L), jnp.float32),
        ],
        mesh=_MESH,
        scratch_types=[
            pltpu.VMEM((CSTEPS, CCH), jnp.int32),
            pltpu.VMEM((CSTEPS, CCH), jnp.int32),
            pltpu.VMEM((CCH, L), jnp.float32),
            pltpu.VMEM((RPT, L), jnp.float32),
            pltpu.VMEM_SHARED((ROWS_PAD, L), jnp.float32),
            pltpu.VMEM_SHARED((ROWS_PAD, L), jnp.float32),
        ],
        compiler_params=_SC_PARAMS,
    )
    return f(gidx, sidx)


def _agg_body(h_hbm, gidx_hbm, sidx_hbm, out_hbm,
              gidx_v, sidx_v, rows_v, zbuf_v, acc_sh, gsem, ssem):
    co = lax.axis_index("c")
    s = lax.axis_index("s")
    w = co * NS + s
    pltpu.sync_copy(gidx_hbm.at[w], gidx_v)
    pltpu.sync_copy(sidx_hbm.at[w], sidx_v)

    @pl.loop(0, 32)
    def _(i):
        for jj in range(D // L):
            zbuf_v[i, pl.ds(jj * L, L)] = jnp.zeros((L,), jnp.float32)

    base = s * RPT

    @pl.loop(0, 19)
    def _(k):
        pltpu.sync_copy(zbuf_v, acc_sh.at[pl.ds(base + k * 32, 32)])

    pltpu.sync_copy(zbuf_v.at[pl.ds(0, RPT - 608)],
                    acc_sh.at[pl.ds(base + 608, RPT - 608)])

    plsc.subcore_barrier()

    for b in range(NB):
        pltpu.async_copy(h_hbm.at[gidx_v.at[b]], rows_v.at[b], gsem.at[b])

    @pl.loop(0, NG)
    def _(g):
        j0 = g * NB
        for b in range(NB):
            pltpu.make_async_copy(
                h_hbm.at[gidx_v.at[j0 + b]], rows_v.at[b], gsem.at[b]
            ).wait()
            pltpu.async_copy(
                rows_v.at[b], acc_sh.at[sidx_v.at[j0 + b]], ssem.at[b],
                add=True)
        for b in range(NB):
            pltpu.make_async_copy(
                rows_v.at[b], acc_sh.at[sidx_v.at[j0 + b]], ssem.at[b]
            ).wait()

            @pl.when(g < NG - 1)
            def _():
                pltpu.async_copy(
                    h_hbm.at[gidx_v.at[j0 + NB + b]], rows_v.at[b], gsem.at[b])

    plsc.subcore_barrier()

    pltpu.sync_copy(acc_sh.at[pl.ds(base, RPT)],
                    out_hbm.at[co, pl.ds(base, RPT)])


def _agg(h, gidx, sidx):
    f = pl.kernel(
        _agg_body,
        out_type=jax.ShapeDtypeStruct((NC, ROWS_PAD, D), jnp.float32),
        mesh=_MESH,
        scratch_types=[
            pltpu.VMEM((STEPS, GCH), jnp.int32),
            pltpu.VMEM((STEPS, GCH), jnp.int32),
            pltpu.VMEM((NB, GCH, D), jnp.float32),
            pltpu.VMEM((32, D), jnp.float32),
            pltpu.VMEM_SHARED((ROWS_PAD, D), jnp.float32),
            pltpu.SemaphoreType.DMA((NB,)),
            pltpu.SemaphoreType.DMA((NB,)),
        ],
        compiler_params=_SC_PARAMS,
    )
    return f(h, gidx, sidx)


def kernel(x, hyperedge_index, W_v2e, b_v2e, W_e2v, b_e2v):
    nidx = hyperedge_index[0].astype(jnp.int32)
    eidx = hyperedge_index[1].astype(jnp.int32)
    pad = P_INC - N_INC
    fill = N_NODES + jnp.arange(pad, dtype=jnp.int32) % (ROWS_PAD - N_NODES)
    nidx_p = jnp.concatenate([nidx, fill])
    eidx_p = jnp.concatenate([eidx, fill])
    nidx_g = nidx_p.reshape(NW, STEPS, GCH)
    eidx_g = eidx_p.reshape(NW, STEPS, GCH)
    nidx_c = nidx_p.reshape(NW, CSTEPS, CCH)
    eidx_c = eidx_p.reshape(NW, CSTEPS, CCH)
    x_p = jnp.pad(x, ((0, ROWS_PAD - N_NODES), (0, 0)))

    h = _linear(x_p, W_v2e.T, b_v2e.reshape(1, D))
    ecnt, vcnt = _counts(nidx_c, eidx_c)
    esum = _agg(h, nidx_g, eidx_g)
    e = _mid(esum, ecnt, W_e2v.T, b_e2v.reshape(1, D))
    vsum = _agg(e, eidx_g, nidx_g)
    out = _final(vsum, vcnt)
    return out[:N_NODES]


# speedup vs baseline: 2.9395x; 1.0381x over previous
import jax
import jax.numpy as jnp
from jax import lax
from jax.experimental import pallas as pl
from jax.experimental.pallas import tpu as pltpu
from jax.experimental.pallas import tpu_sc as plsc

N_NODES = 10000
N_INC = 320000
D = 128

NC = 2
NS = 16
L = 16
NW = NC * NS
GCH = 64
NB = 3
NG = 54
STEPS = NB * NG
P_INC = NW * STEPS * GCH
ROWS_PAD = 10112
RPT = ROWS_PAD // NS
BM = 1264


def _linear_body(x_ref, wt_ref, b_ref, o_ref):
    o_ref[...] = (
        jnp.dot(x_ref[...], wt_ref[...], preferred_element_type=jnp.float32)
        + b_ref[...]
    )


def _linear(x, wt, b):
    m = x.shape[0]
    return pl.pallas_call(
        _linear_body,
        grid=(m // BM,),
        in_specs=[
            pl.BlockSpec((BM, D), lambda i: (i, 0)),
            pl.BlockSpec((D, D), lambda i: (0, 0)),
            pl.BlockSpec((1, D), lambda i: (0, 0)),
        ],
        out_specs=pl.BlockSpec((BM, D), lambda i: (i, 0)),
        out_shape=jax.ShapeDtypeStruct((m, D), jnp.float32),
    )(x, wt, b)


def _mid_body(p_ref, c_ref, wt_ref, b_ref, o_ref):
    s = p_ref[0] + p_ref[1]
    cnt = c_ref[0, :, 0:1] + c_ref[1, :, 0:1]
    mean = jnp.maximum(s / jnp.maximum(cnt, 1.0), 0.0)
    o_ref[...] = (
        jnp.dot(mean, wt_ref[...], preferred_element_type=jnp.float32)
        + b_ref[...]
    )


def _mid(p, c, wt, b):
    return pl.pallas_call(
        _mid_body,
        grid=(ROWS_PAD // BM,),
        in_specs=[
            pl.BlockSpec((NC, BM, D), lambda i: (0, i, 0)),
            pl.BlockSpec((NC, BM, L), lambda i: (0, i, 0)),
            pl.BlockSpec((D, D), lambda i: (0, 0)),
            pl.BlockSpec((1, D), lambda i: (0, 0)),
        ],
        out_specs=pl.BlockSpec((BM, D), lambda i: (i, 0)),
        out_shape=jax.ShapeDtypeStruct((ROWS_PAD, D), jnp.float32),
    )(p, c, wt, b)


def _final_body(q_ref, c_ref, o_ref):
    s = q_ref[0] + q_ref[1]
    cnt = c_ref[0, :, 0:1] + c_ref[1, :, 0:1]
    o_ref[...] = jnp.maximum(s / jnp.maximum(cnt, 1.0), 0.0)


_FBM = 1000


def _final(q, c):
    return pl.pallas_call(
        _final_body,
        grid=(N_NODES // _FBM,),
        in_specs=[
            pl.BlockSpec((NC, _FBM, D), lambda i: (0, i, 0)),
            pl.BlockSpec((NC, _FBM, L), lambda i: (0, i, 0)),
        ],
        out_specs=pl.BlockSpec((_FBM, D), lambda i: (i, 0)),
        out_shape=jax.ShapeDtypeStruct((N_NODES, D), jnp.float32),
    )(q, c)


_MESH = plsc.VectorSubcoreMesh(core_axis_name="c", subcore_axis_name="s")
_SC_PARAMS = pltpu.CompilerParams(use_tc_tiling_on_sc=False)


def _counts_body(gidx_hbm, sidx_hbm, ecnt_hbm, vcnt_hbm,
                 gidx_v, sidx_v, ones_v, zcnt_v, ecnt_sh, vcnt_sh, csem):
    co = lax.axis_index("c")
    s = lax.axis_index("s")
    w = co * NS + s
    pltpu.sync_copy(gidx_hbm.at[w], gidx_v)
    pltpu.sync_copy(sidx_hbm.at[w], sidx_v)

    @pl.loop(0, GCH)
    def _(i):
        ones_v[i, :] = jnp.ones((L,), jnp.float32)

    @pl.loop(0, RPT)
    def _(i):
        zcnt_v[i, :] = jnp.zeros((L,), jnp.float32)

    base = s * RPT
    pltpu.sync_copy(zcnt_v, ecnt_sh.at[pl.ds(base, RPT)])
    pltpu.sync_copy(zcnt_v, vcnt_sh.at[pl.ds(base, RPT)])

    plsc.subcore_barrier()

    @pl.loop(0, STEPS)
    def _(j):
        pltpu.async_copy(ones_v, ecnt_sh.at[sidx_v.at[j]], csem, add=True)
        pltpu.async_copy(ones_v, vcnt_sh.at[gidx_v.at[j]], csem, add=True)

    @pl.loop(0, STEPS)
    def _(j):
        pltpu.make_async_copy(ones_v, ecnt_sh.at[sidx_v.at[j]], csem).wait()
        pltpu.make_async_copy(ones_v, vcnt_sh.at[gidx_v.at[j]], csem).wait()

    plsc.subcore_barrier()

    pltpu.sync_copy(ecnt_sh.at[pl.ds(base, RPT)], ecnt_hbm.at[co, pl.ds(base, RPT)])
    pltpu.sync_copy(vcnt_sh.at[pl.ds(base, RPT)], vcnt_hbm.at[co, pl.ds(base, RPT)])


def _counts(gidx, sidx):
    f = pl.kernel(
        _counts_body,
        out_type=[
            jax.ShapeDtypeStruct((NC, ROWS_PAD, L), jnp.float32),
            jax.ShapeDtypeStruct((NC, ROWS_PAD, L), jnp.float32),
        ],
        mesh=_MESH,
        scratch_types=[
            pltpu.VMEM((STEPS, GCH), jnp.int32),
            pltpu.VMEM((STEPS, GCH), jnp.int32),
            pltpu.VMEM((GCH, L), jnp.float32),
            pltpu.VMEM((RPT, L), jnp.float32),
            pltpu.VMEM_SHARED((ROWS_PAD, L), jnp.float32),
            pltpu.VMEM_SHARED((ROWS_PAD, L), jnp.float32),
            pltpu.SemaphoreType.DMA,
        ],
        compiler_params=_SC_PARAMS,
    )
    return f(gidx, sidx)


def _agg_body(h_hbm, gidx_hbm, sidx_hbm, out_hbm,
              gidx_v, sidx_v, rows_v, zbuf_v, acc_sh, gsem, ssem):
    co = lax.axis_index("c")
    s = lax.axis_index("s")
    w = co * NS + s
    pltpu.sync_copy(gidx_hbm.at[w], gidx_v)
    pltpu.sync_copy(sidx_hbm.at[w], sidx_v)

    @pl.loop(0, 32)
    def _(i):
        for jj in range(D // L):
            zbuf_v[i, pl.ds(jj * L, L)] = jnp.zeros((L,), jnp.float32)

    base = s * RPT

    @pl.loop(0, 19)
    def _(k):
        pltpu.sync_copy(zbuf_v, acc_sh.at[pl.ds(base + k * 32, 32)])

    pltpu.sync_copy(zbuf_v.at[pl.ds(0, RPT - 608)],
                    acc_sh.at[pl.ds(base + 608, RPT - 608)])

    plsc.subcore_barrier()

    for b in range(NB):
        pltpu.async_copy(h_hbm.at[gidx_v.at[b]], rows_v.at[b], gsem.at[b])

    @pl.loop(0, NG)
    def _(g):
        j0 = g * NB
        for b in range(NB):
            pltpu.make_async_copy(
                h_hbm.at[gidx_v.at[j0 + b]], rows_v.at[b], gsem.at[b]
            ).wait()
            pltpu.async_copy(
                rows_v.at[b], acc_sh.at[sidx_v.at[j0 + b]], ssem.at[b],
                add=True)
        for b in range(NB):
            pltpu.make_async_copy(
                rows_v.at[b], acc_sh.at[sidx_v.at[j0 + b]], ssem.at[b]
            ).wait()

            @pl.when(g < NG - 1)
            def _():
                pltpu.async_copy(
                    h_hbm.at[gidx_v.at[j0 + NB + b]], rows_v.at[b], gsem.at[b])

    plsc.subcore_barrier()

    pltpu.sync_copy(acc_sh.at[pl.ds(base, RPT)],
                    out_hbm.at[co, pl.ds(base, RPT)])


def _agg(h, gidx, sidx):
    f = pl.kernel(
        _agg_body,
        out_type=jax.ShapeDtypeStruct((NC, ROWS_PAD, D), jnp.float32),
        mesh=_MESH,
        scratch_types=[
            pltpu.VMEM((STEPS, GCH), jnp.int32),
            pltpu.VMEM((STEPS, GCH), jnp.int32),
            pltpu.VMEM((NB, GCH, D), jnp.float32),
            pltpu.VMEM((32, D), jnp.float32),
            pltpu.VMEM_SHARED((ROWS_PAD, D), jnp.float32),
            pltpu.SemaphoreType.DMA((NB,)),
            pltpu.SemaphoreType.DMA((NB,)),
        ],
        compiler_params=_SC_PARAMS,
    )
    return f(h, gidx, sidx)


def kernel(x, hyperedge_index, W_v2e, b_v2e, W_e2v, b_e2v):
    nidx = hyperedge_index[0].astype(jnp.int32)
    eidx = hyperedge_index[1].astype(jnp.int32)
    pad = P_INC - N_INC
    fill = N_NODES + jnp.arange(pad, dtype=jnp.int32) % (ROWS_PAD - N_NODES)
    nidx_p = jnp.concatenate([nidx, fill])
    eidx_p = jnp.concatenate([eidx, fill])
    nidx_g = nidx_p.reshape(NW, STEPS, GCH)
    eidx_g = eidx_p.reshape(NW, STEPS, GCH)
    x_p = jnp.pad(x, ((0, ROWS_PAD - N_NODES), (0, 0)))

    h = _linear(x_p, W_v2e.T, b_v2e.reshape(1, D))
    ecnt, vcnt = _counts(nidx_g, eidx_g)
    esum = _agg(h, nidx_g, eidx_g)
    e = _mid(esum, ecnt, W_e2v.T, b_e2v.reshape(1, D))
    vsum = _agg(e, eidx_g, nidx_g)
    return _final(vsum, vcnt)


# speedup vs baseline: 3.0950x; 1.0529x over previous
import jax
import jax.numpy as jnp
from jax import lax
from jax.experimental import pallas as pl
from jax.experimental.pallas import tpu as pltpu
from jax.experimental.pallas import tpu_sc as plsc

N_NODES = 10000
N_INC = 320000
D = 128

NC = 2
NS = 16
L = 16
NW = NC * NS
GCH = 48
NB = 4
NG = 54
STEPS = NB * NG
P_INC = NW * STEPS * GCH
ROWS_PAD = 10112
RPT = ROWS_PAD // NS
BM = 1264


def _linear_body(x_ref, wt_ref, b_ref, o_ref):
    o_ref[...] = (
        jnp.dot(x_ref[...], wt_ref[...], preferred_element_type=jnp.float32)
        + b_ref[...]
    )


def _linear(x, wt, b):
    m = x.shape[0]
    return pl.pallas_call(
        _linear_body,
        grid=(m // BM,),
        in_specs=[
            pl.BlockSpec((BM, D), lambda i: (i, 0)),
            pl.BlockSpec((D, D), lambda i: (0, 0)),
            pl.BlockSpec((1, D), lambda i: (0, 0)),
        ],
        out_specs=pl.BlockSpec((BM, D), lambda i: (i, 0)),
        out_shape=jax.ShapeDtypeStruct((m, D), jnp.float32),
    )(x, wt, b)


def _mid_body(p_ref, c_ref, wt_ref, b_ref, o_ref):
    s = p_ref[0] + p_ref[1]
    cnt = c_ref[0, :, 0:1] + c_ref[1, :, 0:1]
    mean = jnp.maximum(s / jnp.maximum(cnt, 1.0), 0.0)
    o_ref[...] = (
        jnp.dot(mean, wt_ref[...], preferred_element_type=jnp.float32)
        + b_ref[...]
    )


def _mid(p, c, wt, b):
    return pl.pallas_call(
        _mid_body,
        grid=(ROWS_PAD // BM,),
        in_specs=[
            pl.BlockSpec((NC, BM, D), lambda i: (0, i, 0)),
            pl.BlockSpec((NC, BM, L), lambda i: (0, i, 0)),
            pl.BlockSpec((D, D), lambda i: (0, 0)),
            pl.BlockSpec((1, D), lambda i: (0, 0)),
        ],
        out_specs=pl.BlockSpec((BM, D), lambda i: (i, 0)),
        out_shape=jax.ShapeDtypeStruct((ROWS_PAD, D), jnp.float32),
    )(p, c, wt, b)


def _final_body(q_ref, c_ref, o_ref):
    s = q_ref[0] + q_ref[1]
    cnt = c_ref[0, :, 0:1] + c_ref[1, :, 0:1]
    o_ref[...] = jnp.maximum(s / jnp.maximum(cnt, 1.0), 0.0)


_FBM = 1000


def _final(q, c):
    return pl.pallas_call(
        _final_body,
        grid=(N_NODES // _FBM,),
        in_specs=[
            pl.BlockSpec((NC, _FBM, D), lambda i: (0, i, 0)),
            pl.BlockSpec((NC, _FBM, L), lambda i: (0, i, 0)),
        ],
        out_specs=pl.BlockSpec((_FBM, D), lambda i: (i, 0)),
        out_shape=jax.ShapeDtypeStruct((N_NODES, D), jnp.float32),
    )(q, c)


_MESH = plsc.VectorSubcoreMesh(core_axis_name="c", subcore_axis_name="s")
_SC_PARAMS = pltpu.CompilerParams(use_tc_tiling_on_sc=False)


def _counts_body(gidx_hbm, sidx_hbm, ecnt_hbm, vcnt_hbm,
                 gidx_v, sidx_v, ones_v, zcnt_v, ecnt_sh, vcnt_sh, csem):
    co = lax.axis_index("c")
    s = lax.axis_index("s")
    w = co * NS + s
    pltpu.sync_copy(gidx_hbm.at[w], gidx_v)
    pltpu.sync_copy(sidx_hbm.at[w], sidx_v)

    @pl.loop(0, GCH)
    def _(i):
        ones_v[i, :] = jnp.ones((L,), jnp.float32)

    @pl.loop(0, RPT)
    def _(i):
        zcnt_v[i, :] = jnp.zeros((L,), jnp.float32)

    base = s * RPT
    pltpu.sync_copy(zcnt_v, ecnt_sh.at[pl.ds(base, RPT)])
    pltpu.sync_copy(zcnt_v, vcnt_sh.at[pl.ds(base, RPT)])

    plsc.subcore_barrier()

    @pl.loop(0, STEPS)
    def _(j):
        pltpu.async_copy(ones_v, ecnt_sh.at[sidx_v.at[j]], csem, add=True)
        pltpu.async_copy(ones_v, vcnt_sh.at[gidx_v.at[j]], csem, add=True)

    @pl.loop(0, STEPS)
    def _(j):
        pltpu.make_async_copy(ones_v, ecnt_sh.at[sidx_v.at[j]], csem).wait()
        pltpu.make_async_copy(ones_v, vcnt_sh.at[gidx_v.at[j]], csem).wait()

    plsc.subcore_barrier()

    pltpu.sync_copy(ecnt_sh.at[pl.ds(base, RPT)], ecnt_hbm.at[co, pl.ds(base, RPT)])
    pltpu.sync_copy(vcnt_sh.at[pl.ds(base, RPT)], vcnt_hbm.at[co, pl.ds(base, RPT)])


def _counts(gidx, sidx):
    f = pl.kernel(
        _counts_body,
        out_type=[
            jax.ShapeDtypeStruct((NC, ROWS_PAD, L), jnp.float32),
            jax.ShapeDtypeStruct((NC, ROWS_PAD, L), jnp.float32),
        ],
        mesh=_MESH,
        scratch_types=[
            pltpu.VMEM((STEPS, GCH), jnp.int32),
            pltpu.VMEM((STEPS, GCH), jnp.int32),
            pltpu.VMEM((GCH, L), jnp.float32),
            pltpu.VMEM((RPT, L), jnp.float32),
            pltpu.VMEM_SHARED((ROWS_PAD, L), jnp.float32),
            pltpu.VMEM_SHARED((ROWS_PAD, L), jnp.float32),
            pltpu.SemaphoreType.DMA,
        ],
        compiler_params=_SC_PARAMS,
    )
    return f(gidx, sidx)


def _agg_body(h_hbm, gidx_hbm, sidx_hbm, out_hbm,
              gidx_v, sidx_v, rows_v, zbuf_v, acc_sh, gsem, ssem):
    co = lax.axis_index("c")
    s = lax.axis_index("s")
    w = co * NS + s
    pltpu.sync_copy(gidx_hbm.at[w], gidx_v)
    pltpu.sync_copy(sidx_hbm.at[w], sidx_v)

    @pl.loop(0, 32)
    def _(i):
        for jj in range(D // L):
            zbuf_v[i, pl.ds(jj * L, L)] = jnp.zeros((L,), jnp.float32)

    base = s * RPT

    @pl.loop(0, 19)
    def _(k):
        pltpu.sync_copy(zbuf_v, acc_sh.at[pl.ds(base + k * 32, 32)])

    pltpu.sync_copy(zbuf_v.at[pl.ds(0, RPT - 608)],
                    acc_sh.at[pl.ds(base + 608, RPT - 608)])

    plsc.subcore_barrier()

    for b in range(NB):
        pltpu.async_copy(h_hbm.at[gidx_v.at[b]], rows_v.at[b], gsem.at[b])

    @pl.loop(0, NG)
    def _(g):
        j0 = g * NB
        for b in range(NB):
            pltpu.make_async_copy(
                h_hbm.at[gidx_v.at[j0 + b]], rows_v.at[b], gsem.at[b]
            ).wait()
            pltpu.async_copy(
                rows_v.at[b], acc_sh.at[sidx_v.at[j0 + b]], ssem.at[b],
                add=True)
        for b in range(NB):
            pltpu.make_async_copy(
                rows_v.at[b], acc_sh.at[sidx_v.at[j0 + b]], ssem.at[b]
            ).wait()

            @pl.when(g < NG - 1)
            def _():
                pltpu.async_copy(
                    h_hbm.at[gidx_v.at[j0 + NB + b]], rows_v.at[b], gsem.at[b])

    plsc.subcore_barrier()

    pltpu.sync_copy(acc_sh.at[pl.ds(base, RPT)],
                    out_hbm.at[co, pl.ds(base, RPT)])


def _agg(h, gidx, sidx):
    f = pl.kernel(
        _agg_body,
        out_type=jax.ShapeDtypeStruct((NC, ROWS_PAD, D), jnp.float32),
        mesh=_MESH,
        scratch_types=[
            pltpu.VMEM((STEPS, GCH), jnp.int32),
            pltpu.VMEM((STEPS, GCH), jnp.int32),
            pltpu.VMEM((NB, GCH, D), jnp.float32),
            pltpu.VMEM((32, D), jnp.float32),
            pltpu.VMEM_SHARED((ROWS_PAD, D), jnp.float32),
            pltpu.SemaphoreType.DMA((NB,)),
            pltpu.SemaphoreType.DMA((NB,)),
        ],
        compiler_params=_SC_PARAMS,
    )
    return f(h, gidx, sidx)


def kernel(x, hyperedge_index, W_v2e, b_v2e, W_e2v, b_e2v):
    nidx = hyperedge_index[0].astype(jnp.int32)
    eidx = hyperedge_index[1].astype(jnp.int32)
    pad = P_INC - N_INC
    fill = N_NODES + jnp.arange(pad, dtype=jnp.int32) % (ROWS_PAD - N_NODES)
    nidx_p = jnp.concatenate([nidx, fill])
    eidx_p = jnp.concatenate([eidx, fill])
    nidx_g = nidx_p.reshape(NW, STEPS, GCH)
    eidx_g = eidx_p.reshape(NW, STEPS, GCH)
    x_p = jnp.pad(x, ((0, ROWS_PAD - N_NODES), (0, 0)))

    h = _linear(x_p, W_v2e.T, b_v2e.reshape(1, D))
    ecnt, vcnt = _counts(nidx_g, eidx_g)
    esum = _agg(h, nidx_g, eidx_g)
    e = _mid(esum, ecnt, W_e2v.T, b_e2v.reshape(1, D))
    vsum = _agg(e, eidx_g, nidx_g)
    return _final(vsum, vcnt)


# speedup vs baseline: 3.1080x; 1.0042x over previous
import jax
import jax.numpy as jnp
from jax import lax
from jax.experimental import pallas as pl
from jax.experimental.pallas import tpu as pltpu
from jax.experimental.pallas import tpu_sc as plsc

N_NODES = 10000
N_INC = 320000
D = 128

NC = 2
NS = 16
L = 16
NW = NC * NS
GCH = 32
NB = 6
NG = 54
STEPS = NB * NG
P_INC = NW * STEPS * GCH
ROWS_PAD = 10112
RPT = ROWS_PAD // NS
BM = 1264


def _linear_body(x_ref, wt_ref, b_ref, o_ref):
    o_ref[...] = (
        jnp.dot(x_ref[...], wt_ref[...], preferred_element_type=jnp.float32)
        + b_ref[...]
    )


def _linear(x, wt, b):
    m = x.shape[0]
    return pl.pallas_call(
        _linear_body,
        grid=(m // BM,),
        in_specs=[
            pl.BlockSpec((BM, D), lambda i: (i, 0)),
            pl.BlockSpec((D, D), lambda i: (0, 0)),
            pl.BlockSpec((1, D), lambda i: (0, 0)),
        ],
        out_specs=pl.BlockSpec((BM, D), lambda i: (i, 0)),
        out_shape=jax.ShapeDtypeStruct((m, D), jnp.float32),
    )(x, wt, b)


def _mid_body(p_ref, c_ref, wt_ref, b_ref, o_ref):
    s = p_ref[0] + p_ref[1]
    cnt = c_ref[0, :, 0:1] + c_ref[1, :, 0:1]
    mean = jnp.maximum(s / jnp.maximum(cnt, 1.0), 0.0)
    o_ref[...] = (
        jnp.dot(mean, wt_ref[...], preferred_element_type=jnp.float32)
        + b_ref[...]
    )


def _mid(p, c, wt, b):
    return pl.pallas_call(
        _mid_body,
        grid=(ROWS_PAD // BM,),
        in_specs=[
            pl.BlockSpec((NC, BM, D), lambda i: (0, i, 0)),
            pl.BlockSpec((NC, BM, L), lambda i: (0, i, 0)),
            pl.BlockSpec((D, D), lambda i: (0, 0)),
            pl.BlockSpec((1, D), lambda i: (0, 0)),
        ],
        out_specs=pl.BlockSpec((BM, D), lambda i: (i, 0)),
        out_shape=jax.ShapeDtypeStruct((ROWS_PAD, D), jnp.float32),
    )(p, c, wt, b)


def _final_body(q_ref, c_ref, o_ref):
    s = q_ref[0] + q_ref[1]
    cnt = c_ref[0, :, 0:1] + c_ref[1, :, 0:1]
    o_ref[...] = jnp.maximum(s / jnp.maximum(cnt, 1.0), 0.0)


_FBM = 1000


def _final(q, c):
    return pl.pallas_call(
        _final_body,
        grid=(N_NODES // _FBM,),
        in_specs=[
            pl.BlockSpec((NC, _FBM, D), lambda i: (0, i, 0)),
            pl.BlockSpec((NC, _FBM, L), lambda i: (0, i, 0)),
        ],
        out_specs=pl.BlockSpec((_FBM, D), lambda i: (i, 0)),
        out_shape=jax.ShapeDtypeStruct((N_NODES, D), jnp.float32),
    )(q, c)


_MESH = plsc.VectorSubcoreMesh(core_axis_name="c", subcore_axis_name="s")
_SC_PARAMS = pltpu.CompilerParams(use_tc_tiling_on_sc=False)


def _counts_body(gidx_hbm, sidx_hbm, ecnt_hbm, vcnt_hbm,
                 gidx_v, sidx_v, ones_v, zcnt_v, ecnt_sh, vcnt_sh, csem):
    co = lax.axis_index("c")
    s = lax.axis_index("s")
    w = co * NS + s
    pltpu.sync_copy(gidx_hbm.at[w], gidx_v)
    pltpu.sync_copy(sidx_hbm.at[w], sidx_v)

    @pl.loop(0, GCH)
    def _(i):
        ones_v[i, :] = jnp.ones((L,), jnp.float32)

    @pl.loop(0, RPT)
    def _(i):
        zcnt_v[i, :] = jnp.zeros((L,), jnp.float32)

    base = s * RPT
    pltpu.sync_copy(zcnt_v, ecnt_sh.at[pl.ds(base, RPT)])
    pltpu.sync_copy(zcnt_v, vcnt_sh.at[pl.ds(base, RPT)])

    plsc.subcore_barrier()

    @pl.loop(0, STEPS)
    def _(j):
        pltpu.async_copy(ones_v, ecnt_sh.at[sidx_v.at[j]], csem, add=True)
        pltpu.async_copy(ones_v, vcnt_sh.at[gidx_v.at[j]], csem, add=True)

    @pl.loop(0, STEPS)
    def _(j):
        pltpu.make_async_copy(ones_v, ecnt_sh.at[sidx_v.at[j]], csem).wait()
        pltpu.make_async_copy(ones_v, vcnt_sh.at[gidx_v.at[j]], csem).wait()

    plsc.subcore_barrier()

    pltpu.sync_copy(ecnt_sh.at[pl.ds(base, RPT)], ecnt_hbm.at[co, pl.ds(base, RPT)])
    pltpu.sync_copy(vcnt_sh.at[pl.ds(base, RPT)], vcnt_hbm.at[co, pl.ds(base, RPT)])


def _counts(gidx, sidx):
    f = pl.kernel(
        _counts_body,
        out_type=[
            jax.ShapeDtypeStruct((NC, ROWS_PAD, L), jnp.float32),
            jax.ShapeDtypeStruct((NC, ROWS_PAD, L), jnp.float32),
        ],
        mesh=_MESH,
        scratch_types=[
            pltpu.VMEM((STEPS, GCH), jnp.int32),
            pltpu.VMEM((STEPS, GCH), jnp.int32),
            pltpu.VMEM((GCH, L), jnp.float32),
            pltpu.VMEM((RPT, L), jnp.float32),
            pltpu.VMEM_SHARED((ROWS_PAD, L), jnp.float32),
            pltpu.VMEM_SHARED((ROWS_PAD, L), jnp.float32),
            pltpu.SemaphoreType.DMA,
        ],
        compiler_params=_SC_PARAMS,
    )
    return f(gidx, sidx)


def _agg_body(h_hbm, gidx_hbm, sidx_hbm, out_hbm,
              gidx_v, sidx_v, rows_v, zbuf_v, acc_sh, gsem, ssem):
    co = lax.axis_index("c")
    s = lax.axis_index("s")
    w = co * NS + s
    pltpu.sync_copy(gidx_hbm.at[w], gidx_v)
    pltpu.sync_copy(sidx_hbm.at[w], sidx_v)

    @pl.loop(0, 32)
    def _(i):
        for jj in range(D // L):
            zbuf_v[i, pl.ds(jj * L, L)] = jnp.zeros((L,), jnp.float32)

    base = s * RPT

    @pl.loop(0, 19)
    def _(k):
        pltpu.sync_copy(zbuf_v, acc_sh.at[pl.ds(base + k * 32, 32)])

    pltpu.sync_copy(zbuf_v.at[pl.ds(0, RPT - 608)],
                    acc_sh.at[pl.ds(base + 608, RPT - 608)])

    plsc.subcore_barrier()

    for b in range(NB):
        pltpu.async_copy(h_hbm.at[gidx_v.at[b]], rows_v.at[b], gsem.at[b])

    @pl.loop(0, NG)
    def _(g):
        j0 = g * NB
        for b in range(NB):
            pltpu.make_async_copy(
                h_hbm.at[gidx_v.at[j0 + b]], rows_v.at[b], gsem.at[b]
            ).wait()
            pltpu.async_copy(
                rows_v.at[b], acc_sh.at[sidx_v.at[j0 + b]], ssem.at[b],
                add=True)
        for b in range(NB):
            pltpu.make_async_copy(
                rows_v.at[b], acc_sh.at[sidx_v.at[j0 + b]], ssem.at[b]
            ).wait()

            @pl.when(g < NG - 1)
            def _():
                pltpu.async_copy(
                    h_hbm.at[gidx_v.at[j0 + NB + b]], rows_v.at[b], gsem.at[b])

    plsc.subcore_barrier()

    pltpu.sync_copy(acc_sh.at[pl.ds(base, RPT)],
                    out_hbm.at[co, pl.ds(base, RPT)])


def _agg(h, gidx, sidx):
    f = pl.kernel(
        _agg_body,
        out_type=jax.ShapeDtypeStruct((NC, ROWS_PAD, D), jnp.float32),
        mesh=_MESH,
        scratch_types=[
            pltpu.VMEM((STEPS, GCH), jnp.int32),
            pltpu.VMEM((STEPS, GCH), jnp.int32),
            pltpu.VMEM((NB, GCH, D), jnp.float32),
            pltpu.VMEM((32, D), jnp.float32),
            pltpu.VMEM_SHARED((ROWS_PAD, D), jnp.float32),
            pltpu.SemaphoreType.DMA((NB,)),
            pltpu.SemaphoreType.DMA((NB,)),
        ],
        compiler_params=_SC_PARAMS,
    )
    return f(h, gidx, sidx)


def kernel(x, hyperedge_index, W_v2e, b_v2e, W_e2v, b_e2v):
    nidx = hyperedge_index[0].astype(jnp.int32)
    eidx = hyperedge_index[1].astype(jnp.int32)
    pad = P_INC - N_INC
    fill = N_NODES + jnp.arange(pad, dtype=jnp.int32) % (ROWS_PAD - N_NODES)
    nidx_p = jnp.concatenate([nidx, fill])
    eidx_p = jnp.concatenate([eidx, fill])
    nidx_g = nidx_p.reshape(NW, STEPS, GCH)
    eidx_g = eidx_p.reshape(NW, STEPS, GCH)
    x_p = jnp.pad(x, ((0, ROWS_PAD - N_NODES), (0, 0)))

    h = _linear(x_p, W_v2e.T, b_v2e.reshape(1, D))
    ecnt, vcnt = _counts(nidx_g, eidx_g)
    esum = _agg(h, nidx_g, eidx_g)
    e = _mid(esum, ecnt, W_e2v.T, b_e2v.reshape(1, D))
    vsum = _agg(e, eidx_g, nidx_g)
    return _final(vsum, vcnt)
